# baseline (speedup 1.0000x reference)
import functools

import jax
import jax.numpy as jnp
from jax import lax
from jax.experimental import pallas as pl
from jax.experimental.pallas import tpu as pltpu
from jax.experimental.pallas import tpu_sc as plsc

VOCAB_SIZE = 100000
EMB_D = 32
CTX = 20
HID = 30
B = 1024

_NC = 2
_NS = 16
_NW = _NC * _NS

_TILE_V = 4096


def _make_sc_gather(num_rows: int):
    rows_per_w = num_rows // _NW
    batch_per_w = rows_per_w // CTX
    out_w = rows_per_w * EMB_D
    mesh = plsc.VectorSubcoreMesh(core_axis_name="c", subcore_axis_name="s")

    @functools.partial(
        pl.kernel,
        mesh=mesh,
        compiler_params=pltpu.CompilerParams(needs_layout_passes=False),
        out_type=jax.ShapeDtypeStruct((B, CTX * EMB_D), jnp.float32),
        scratch_types=[
            pltpu.VMEM((rows_per_w,), jnp.int32),
            pltpu.VMEM((rows_per_w,), jnp.int32),
            pltpu.VMEM((rows_per_w, 128), jnp.float32),
            pltpu.VMEM((batch_per_w, CTX * EMB_D), jnp.float32),
            pltpu.SemaphoreType.DMA,
        ],
    )
    def gather_kernel(idx_hbm, table4_hbm, out_hbm, idx_v, idx4_v, rows4_v,
                      emb_v, sem):
        wid = lax.axis_index("s") * _NC + lax.axis_index("c")
        base = wid * rows_per_w
        pltpu.sync_copy(idx_hbm.at[pl.ds(base, rows_per_w)], idx_v)

        two = jnp.full((16,), 2, jnp.int32)
        three = jnp.full((16,), 3, jnp.int32)

        def shift_body(j, _):
            sl = pl.ds(j * 16, 16)
            idx4_v[sl] = lax.shift_right_logical(idx_v[sl], two)
            return 0

        lax.fori_loop(0, rows_per_w // 16, shift_body, 0)
        pltpu.async_copy(table4_hbm.at[idx4_v], rows4_v, sem).wait()

        lane = lax.iota(jnp.int32, 16)
        width = jnp.full((16,), CTX * EMB_D, jnp.int32)

        def select_body(j, _):
            row16 = j * 16 + lane
            r16 = idx_v[pl.ds(j * 16, 16)]
            off = lax.bitwise_and(r16, three) * 32
            for d in range(EMB_D):
                vals = plsc.load_gather(rows4_v, [row16, off + d])
                flat = row16 * EMB_D + d
                plsc.store_scatter(
                    emb_v, [lax.div(flat, width), lax.rem(flat, width)],
                    vals)
            return 0

        lax.fori_loop(0, rows_per_w // 16, select_body, 0)
        pltpu.sync_copy(emb_v, out_hbm.at[pl.ds(wid * batch_per_w,
                                                batch_per_w)])

    return gather_kernel


def _mlp_body(emb_ref, w1_ref, b1_ref, w2_ref, b2_ref, out_ref, hid_ref):
    @pl.when(pl.program_id(0) == 0)
    def _():
        h = lax.dot_general(
            emb_ref[...], w1_ref[...],
            (((1,), (1,)), ((), ())),
            preferred_element_type=jnp.float32,
        )
        hid_ref[...] = jnp.tanh(h + b1_ref[...])

    y = lax.dot_general(
        hid_ref[...], w2_ref[...],
        (((1,), (0,)), ((), ())),
        preferred_element_type=jnp.float32,
    )
    out_ref[...] = y + b2_ref[...]


def kernel(inputs, emb, W1, b1, W2, b2):
    idx = inputs.reshape(-1).astype(jnp.int32)
    table4 = emb.reshape(VOCAB_SIZE // 4, 4 * EMB_D)
    embeds = _make_sc_gather(idx.shape[0])(idx, table4)

    grid = (pl.cdiv(VOCAB_SIZE, _TILE_V),)
    y = pl.pallas_call(
        _mlp_body,
        grid=grid,
        in_specs=[
            pl.BlockSpec((B, CTX * EMB_D), lambda j: (0, 0)),
            pl.BlockSpec((HID, CTX * EMB_D), lambda j: (0, 0)),
            pl.BlockSpec((1, HID), lambda j: (0, 0)),
            pl.BlockSpec((HID, _TILE_V), lambda j: (0, j)),
            pl.BlockSpec((1, _TILE_V), lambda j: (0, j)),
        ],
        out_specs=pl.BlockSpec((B, _TILE_V), lambda j: (0, j)),
        out_shape=jax.ShapeDtypeStruct((B, VOCAB_SIZE), jnp.float32),
        scratch_shapes=[pltpu.VMEM((B, HID), jnp.float32)],
    )(embeds, W1, b1.reshape(1, HID), W2.T, b2.reshape(1, VOCAB_SIZE))
    return y


# speedup vs baseline: 2.5495x; 2.5495x over previous
import functools

import jax
import jax.numpy as jnp
from jax import lax
from jax.experimental import pallas as pl
from jax.experimental.pallas import tpu as pltpu
from jax.experimental.pallas import tpu_sc as plsc

VOCAB_SIZE = 100000
EMB_D = 32
CTX = 20
HID = 30
B = 1024

_NC = 2
_NS = 16
_NW = _NC * _NS

_TILE_V = 2048


def _make_sc_gather(num_rows: int):
    rows_per_w = num_rows // _NW
    batch_per_w = rows_per_w // CTX
    out_w = rows_per_w * EMB_D
    mesh = plsc.VectorSubcoreMesh(core_axis_name="c", subcore_axis_name="s")

    @functools.partial(
        pl.kernel,
        mesh=mesh,
        compiler_params=pltpu.CompilerParams(needs_layout_passes=False),
        out_type=jax.ShapeDtypeStruct((B, CTX * EMB_D), jnp.float32),
        scratch_types=[
            pltpu.VMEM((rows_per_w,), jnp.int32),
            pltpu.VMEM((rows_per_w,), jnp.int32),
            pltpu.VMEM((rows_per_w, 128), jnp.float32),
            pltpu.VMEM((batch_per_w, CTX * EMB_D), jnp.float32),
            pltpu.SemaphoreType.DMA,
        ],
    )
    def gather_kernel(idx_hbm, table4_hbm, out_hbm, idx_v, idx4_v, rows4_v,
                      emb_v, sem):
        wid = lax.axis_index("s") * _NC + lax.axis_index("c")
        base = wid * rows_per_w
        pltpu.sync_copy(idx_hbm.at[pl.ds(base, rows_per_w)], idx_v)

        two = jnp.full((16,), 2, jnp.int32)
        three = jnp.full((16,), 3, jnp.int32)

        def shift_body(j, _):
            sl = pl.ds(j * 16, 16)
            idx4_v[sl] = lax.shift_right_logical(idx_v[sl], two)
            return 0

        lax.fori_loop(0, rows_per_w // 16, shift_body, 0)
        pltpu.async_copy(table4_hbm.at[idx4_v], rows4_v, sem).wait()

        lane = lax.iota(jnp.int32, 16)
        width = jnp.full((16,), CTX * EMB_D, jnp.int32)

        def select_body(j, _):
            row16 = j * 16 + lane
            r16 = idx_v[pl.ds(j * 16, 16)]
            off = lax.bitwise_and(r16, three) * 32
            for d in range(EMB_D):
                vals = plsc.load_gather(rows4_v, [row16, off + d])
                flat = row16 * EMB_D + d
                plsc.store_scatter(
                    emb_v, [lax.div(flat, width), lax.rem(flat, width)],
                    vals)
            return 0

        lax.fori_loop(0, rows_per_w // 16, select_body, 0)
        pltpu.sync_copy(emb_v, out_hbm.at[pl.ds(wid * batch_per_w,
                                                batch_per_w)])

    return gather_kernel


def _mlp_body(emb_ref, w1_ref, b1_ref, w2t_ref, b2_ref, out_ref, hid_ref,
              w2s_ref):
    @pl.when(pl.program_id(0) == 0)
    def _():
        h = lax.dot_general(
            w1_ref[...], emb_ref[...],
            (((1,), (1,)), ((), ())),
            preferred_element_type=jnp.float32,
        )
        hid_ref[0:HID, :] = jnp.tanh(h + b1_ref[...])
        hid_ref[HID:HID + 1, :] = jnp.ones((1, B), jnp.float32)
        hid_ref[HID + 1:, :] = jnp.zeros((1, B), jnp.float32)
        w2s_ref[HID + 1:, :] = jnp.zeros((1, _TILE_V), jnp.float32)

    w2s_ref[0:HID, :] = w2t_ref[...]
    w2s_ref[HID:HID + 1, :] = b2_ref[...]
    out_ref[...] = lax.dot_general(
        w2s_ref[...], hid_ref[...],
        (((0,), (0,)), ((), ())),
        preferred_element_type=jnp.float32,
    )


def kernel(inputs, emb, W1, b1, W2, b2):
    idx = inputs.reshape(-1).astype(jnp.int32)
    table4 = emb.reshape(VOCAB_SIZE // 4, 4 * EMB_D)
    embeds = _make_sc_gather(idx.shape[0])(idx, table4)

    grid = (pl.cdiv(VOCAB_SIZE, _TILE_V),)
    yt = pl.pallas_call(
        _mlp_body,
        grid=grid,
        in_specs=[
            pl.BlockSpec((B, CTX * EMB_D), lambda j: (0, 0)),
            pl.BlockSpec((HID, CTX * EMB_D), lambda j: (0, 0)),
            pl.BlockSpec((HID, 1), lambda j: (0, 0)),
            pl.BlockSpec((HID, _TILE_V), lambda j: (0, j)),
            pl.BlockSpec((1, _TILE_V), lambda j: (0, j)),
        ],
        out_specs=pl.BlockSpec((_TILE_V, B), lambda j: (j, 0)),
        out_shape=jax.ShapeDtypeStruct((VOCAB_SIZE, B), jnp.float32),
        scratch_shapes=[
            pltpu.VMEM((HID + 2, B), jnp.float32),
            pltpu.VMEM((HID + 2, _TILE_V), jnp.float32),
        ],
    )(embeds, W1, b1.reshape(HID, 1), W2.T, b2.reshape(1, VOCAB_SIZE))
    return yt.T


# speedup vs baseline: 2.8367x; 1.1127x over previous
import functools

import jax
import jax.numpy as jnp
from jax import lax
from jax.experimental import pallas as pl
from jax.experimental.pallas import tpu as pltpu
from jax.experimental.pallas import tpu_sc as plsc

VOCAB_SIZE = 100000
EMB_D = 32
CTX = 20
HID = 30
B = 1024

_NC = 2
_NS = 16
_NW = _NC * _NS

_TILE_V = 2048


def _make_sc_gather(num_rows: int):
    rows_per_w = num_rows // _NW
    batch_per_w = rows_per_w // CTX
    mesh = plsc.VectorSubcoreMesh(core_axis_name="c", subcore_axis_name="s")

    @functools.partial(
        pl.kernel,
        mesh=mesh,
        compiler_params=pltpu.CompilerParams(
            needs_layout_passes=False, use_tc_tiling_on_sc=False),
        out_type=jax.ShapeDtypeStruct((num_rows, EMB_D), jnp.float32),
        scratch_types=[
            pltpu.VMEM((rows_per_w,), jnp.int32),
            pltpu.VMEM((rows_per_w, EMB_D), jnp.float32),
            pltpu.SemaphoreType.DMA,
        ],
    )
    def gather_kernel(idx_hbm, table_hbm, out_hbm, idx_v, rows_v, sem):
        wid = lax.axis_index("s") * _NC + lax.axis_index("c")
        base = wid * rows_per_w
        pltpu.sync_copy(idx_hbm.at[pl.ds(base, rows_per_w)], idx_v)
        pltpu.async_copy(table_hbm.at[idx_v], rows_v, sem).wait()
        pltpu.sync_copy(rows_v, out_hbm.at[pl.ds(base, rows_per_w)])

    return gather_kernel


def _mlp_body(emb_ref, w1_ref, b1_ref, w2t_ref, b2_ref, out_ref, hid_ref,
              w2s_ref):
    @pl.when(pl.program_id(0) == 0)
    def _():
        h = lax.dot_general(
            w1_ref[...], emb_ref[...],
            (((1,), (1,)), ((), ())),
            preferred_element_type=jnp.float32,
        )
        hid_ref[0:HID, :] = jnp.tanh(h + b1_ref[...])
        hid_ref[HID:HID + 1, :] = jnp.ones((1, B), jnp.float32)
        hid_ref[HID + 1:, :] = jnp.zeros((1, B), jnp.float32)
        w2s_ref[HID + 1:, :] = jnp.zeros((1, _TILE_V), jnp.float32)

    w2s_ref[0:HID, :] = w2t_ref[...]
    w2s_ref[HID:HID + 1, :] = b2_ref[...]
    out_ref[...] = lax.dot_general(
        w2s_ref[...], hid_ref[...],
        (((0,), (0,)), ((), ())),
        preferred_element_type=jnp.float32,
    )


def kernel(inputs, emb, W1, b1, W2, b2):
    idx = inputs.reshape(-1).astype(jnp.int32)
    gathered = _make_sc_gather(idx.shape[0])(idx, emb)
    embeds = gathered.reshape(B, CTX * EMB_D)

    grid = (pl.cdiv(VOCAB_SIZE, _TILE_V),)
    yt = pl.pallas_call(
        _mlp_body,
        grid=grid,
        in_specs=[
            pl.BlockSpec((B, CTX * EMB_D), lambda j: (0, 0)),
            pl.BlockSpec((HID, CTX * EMB_D), lambda j: (0, 0)),
            pl.BlockSpec((HID, 1), lambda j: (0, 0)),
            pl.BlockSpec((HID, _TILE_V), lambda j: (0, j)),
            pl.BlockSpec((1, _TILE_V), lambda j: (0, j)),
        ],
        out_specs=pl.BlockSpec((_TILE_V, B), lambda j: (j, 0)),
        out_shape=jax.ShapeDtypeStruct((VOCAB_SIZE, B), jnp.float32),
        scratch_shapes=[
            pltpu.VMEM((HID + 2, B), jnp.float32),
            pltpu.VMEM((HID + 2, _TILE_V), jnp.float32),
        ],
    )(embeds, W1, b1.reshape(HID, 1), W2.T, b2.reshape(1, VOCAB_SIZE))
    return yt.T
